# baseline (speedup 1.0000x reference)
import functools

import jax
import jax.numpy as jnp
from jax import lax
from jax.experimental import pallas as pl
from jax.experimental.pallas import tpu as pltpu
from jax.experimental.pallas import tpu_sc as plsc

_DIM = 128
_ROWS = 1000
_ROWS_PAD = 1024
_BATCH = 16384
_SPLIT = 6144
_CHUNK = 64
_TBLK = 512


@functools.lru_cache(maxsize=None)
def _build_sc_gather():
    info = plsc.get_sparse_core_info()
    nw = info.num_cores * info.num_subcores
    bpw = _SPLIT // nw
    nchunks = bpw // _CHUNK
    mesh = plsc.VectorSubcoreMesh(core_axis_name="c", subcore_axis_name="s")

    @functools.partial(
        pl.kernel,
        mesh=mesh,
        out_type=jax.ShapeDtypeStruct((_BATCH, _DIM), jnp.float32),
        scratch_types=[
            pltpu.VMEM((bpw,), jnp.int32),
            pltpu.VMEM((_CHUNK, _DIM), jnp.float32),
            pltpu.VMEM((_CHUNK, _DIM), jnp.float32),
            pltpu.VMEM_SHARED((_ROWS, _DIM), jnp.float32),
            pltpu.SemaphoreType.DMA,
            pltpu.SemaphoreType.DMA,
        ],
    )
    def gather(idx_hbm, table_hbm, out_hbm, idx_v, r0, r1, table_sp, gsem, ssem):
        sid = lax.axis_index("s")
        wid = sid * info.num_cores + lax.axis_index("c")
        base = wid * bpw
        ih = pltpu.async_copy(idx_hbm.at[pl.ds(base, bpw)], idx_v, ssem)

        @pl.when(sid < 15)
        def _():
            pltpu.sync_copy(table_hbm.at[pl.ds(sid * 64, 64)],
                            table_sp.at[pl.ds(sid * 64, 64)])

        @pl.when(sid == 15)
        def _():
            pltpu.sync_copy(table_hbm.at[pl.ds(960, 40)],
                            table_sp.at[pl.ds(960, 40)])

        ih.wait()
        plsc.subcore_barrier()
        bufs = (r0, r1)

        def g(j):
            return pltpu.async_copy(
                table_sp.at[idx_v.at[pl.ds(j * _CHUNK, _CHUNK)]],
                bufs[j % 2], gsem)

        def s(j):
            return pltpu.async_copy(
                bufs[j % 2],
                out_hbm.at[pl.ds(base + j * _CHUNK, _CHUNK)], ssem)

        gh = [None] * nchunks
        sh = [None] * nchunks
        gh[0] = g(0)
        for j in range(nchunks):
            gh[j].wait()
            if j + 1 < nchunks:
                if j >= 1:
                    sh[j - 1].wait()
                gh[j + 1] = g(j + 1)
            sh[j] = s(j)
        sh[nchunks - 2].wait()
        sh[nchunks - 1].wait()

    return gather


def _tc_body(idx_ref, hi_ref, lo_ref, out_ref):
    idx = idx_ref[0, 0, :]
    iot = lax.broadcasted_iota(jnp.int32, (_ROWS_PAD, _TBLK), 0)
    oh = (iot == idx[None, :]).astype(jnp.bfloat16)
    dn = (((0,), (0,)), ((), ()))
    acc = lax.dot_general(oh, hi_ref[...], dn,
                          preferred_element_type=jnp.float32)
    acc = acc + lax.dot_general(oh, lo_ref[...], dn,
                                preferred_element_type=jnp.float32)
    out_ref[...] = acc


@functools.lru_cache(maxsize=None)
def _build_tc_onehot():
    nblk = (_BATCH - _SPLIT) // _TBLK
    return pl.pallas_call(
        _tc_body,
        grid=(nblk,),
        in_specs=[
            pl.BlockSpec((1, 1, _TBLK), lambda j: (j, 0, 0)),
            pl.BlockSpec((_ROWS_PAD, _DIM), lambda j: (0, 0)),
            pl.BlockSpec((_ROWS_PAD, _DIM), lambda j: (0, 0)),
        ],
        out_specs=pl.BlockSpec((_TBLK, _DIM), lambda j: (j, 0)),
        out_shape=jax.ShapeDtypeStruct((_BATCH - _SPLIT, _DIM), jnp.float32),
    )


@jax.jit
def kernel(timesteps, pe):
    idx = timesteps.astype(jnp.int32)
    out_sc = _build_sc_gather()(idx, pe)

    pe_pad = jnp.pad(pe, ((0, _ROWS_PAD - _ROWS), (0, 0)))
    hi = pe_pad.astype(jnp.bfloat16)
    lo = (pe_pad - hi.astype(jnp.float32)).astype(jnp.bfloat16)
    idx_tail = idx[_SPLIT:].reshape((_BATCH - _SPLIT) // _TBLK, 1, _TBLK)
    out_tc = _build_tc_onehot()(idx_tail, hi, lo)

    return lax.dynamic_update_slice(out_sc, out_tc, (_SPLIT, 0))


# speedup vs baseline: 1.6611x; 1.6611x over previous
import functools

import jax
import jax.numpy as jnp
from jax import lax
from jax.experimental import pallas as pl
from jax.experimental.pallas import tpu as pltpu
from jax.experimental.pallas import tpu_sc as plsc

_DIM = 128
_ROWS = 1000
_BATCH = 16384
_CHUNK = 128


@functools.lru_cache(maxsize=None)
def _build_gather():
    info = plsc.get_sparse_core_info()
    nw = info.num_cores * info.num_subcores
    bpw = _BATCH // nw
    nchunks = bpw // _CHUNK
    mesh = plsc.VectorSubcoreMesh(core_axis_name="c", subcore_axis_name="s")

    @functools.partial(
        pl.kernel,
        mesh=mesh,
        out_type=jax.ShapeDtypeStruct((_BATCH, _DIM), jnp.float32),
        scratch_types=[
            pltpu.VMEM((bpw,), jnp.int32),
            pltpu.VMEM((_CHUNK, _DIM), jnp.float32),
            pltpu.VMEM((_CHUNK, _DIM), jnp.float32),
            pltpu.VMEM((_CHUNK, _DIM), jnp.float32),
            pltpu.VMEM((_CHUNK, _DIM), jnp.float32),
            pltpu.VMEM_SHARED((_ROWS, _DIM), jnp.float32),
            pltpu.SemaphoreType.DMA,
            pltpu.SemaphoreType.DMA,
        ],
    )
    def gather(idx_hbm, table_hbm, out_hbm, idx_v, r0, r1, r2, r3, table_sp,
               gsem, ssem):
        sid = lax.axis_index("s")
        wid = sid * info.num_cores + lax.axis_index("c")
        base = wid * bpw
        ih = pltpu.async_copy(idx_hbm.at[pl.ds(base, bpw)], idx_v, ssem)

        @pl.when(sid < 15)
        def _():
            pltpu.sync_copy(table_hbm.at[pl.ds(sid * 64, 64)],
                            table_sp.at[pl.ds(sid * 64, 64)])

        @pl.when(sid == 15)
        def _():
            pltpu.sync_copy(table_hbm.at[pl.ds(960, 40)],
                            table_sp.at[pl.ds(960, 40)])

        ih.wait()
        plsc.subcore_barrier()
        bufs = (r0, r1, r2, r3)

        def g(j):
            return pltpu.async_copy(
                table_sp.at[idx_v.at[pl.ds(j * _CHUNK, _CHUNK)]],
                bufs[j], gsem)

        def s(j):
            return pltpu.async_copy(
                bufs[j],
                out_hbm.at[pl.ds(base + j * _CHUNK, _CHUNK)], ssem)

        gh = [g(j) for j in range(nchunks)]
        sh = []
        for j in range(nchunks):
            gh[j].wait()
            sh.append(s(j))
        for h in sh:
            h.wait()

    return gather


@jax.jit
def kernel(timesteps, pe):
    return _build_gather()(timesteps.astype(jnp.int32), pe)
